# Initial kernel scaffold; baseline (speedup 1.0000x reference)
import jax
import jax.numpy as jnp
from jax.experimental import pallas as pl


def kernel(node_feats, edge_feats, candidate_scores, src, dst, graph_ids, pin_W, pin_b, pcm1_W, pcm1_b, upd1_W, upd1_b, pcm2_W, pcm2_b, upd2_W, upd2_b, fc1_W, fc1_b, fc2_W, fc2_b):
    raise NotImplementedError("write your pallas kernel here")


# speedup vs baseline: 2.3259x; 2.3259x over previous
import jax
import jax.numpy as jnp
from jax.experimental import pallas as pl
from jax.experimental.pallas import tpu as pltpu

_NG = 100
_NP = 100
_EP = 3200
_DN = 128
_DE = 16
_H = 128


def _dot(a, b):
    return jax.lax.dot_general(a, b, (((1,), (0,)), ((), ())),
                               precision=jax.lax.Precision.HIGHEST,
                               preferred_element_type=jnp.float32)


def _body(nf_ref, ef_ref, srcl_ref, dstl_ref,
          pinW_ref, pinb_ref, pcm1W_ref, pcm1b_ref, upd1W_ref, upd1b_ref,
          pcm2W_ref, pcm2b_ref, upd2W_ref, upd2b_ref,
          fc1W_ref, fc1b_ref, fc2W_ref, fc2b_ref,
          out_ref, react_scr):
    g = pl.program_id(0)
    f32 = jnp.float32

    nf = nf_ref[0]
    ef = ef_ref[0]
    sl = srcl_ref[0]
    dl = dstl_ref[0]

    Gs = (jax.lax.broadcasted_iota(jnp.int32, (_EP, _NP), 1) == sl).astype(f32)
    Gd = (jax.lax.broadcasted_iota(jnp.int32, (_NP, _EP), 0) == dl).astype(f32)

    pinW = pinW_ref[...]
    pcm1W = pcm1W_ref[...]
    upd1W = upd1W_ref[...]
    pcm2W = pcm2W_ref[...]
    upd2W = upd2W_ref[...]

    ep1 = _dot(ef, pcm1W[_H:]) + pcm1b_ref[...]
    ep2 = _dot(ef, pcm2W[_H:]) + pcm2b_ref[...]

    h = jnp.maximum(_dot(nf, pinW) + pinb_ref[...], 0.0)
    for _ in range(3):
        hp = _dot(h, pcm1W[:_H])
        msg = jnp.maximum(_dot(Gs, hp) + ep1, 0.0)
        agg = _dot(Gd, msg)
        h = jnp.maximum(_dot(h, upd1W[:_H]) + _dot(agg, upd1W[_H:])
                        + upd1b_ref[...], 0.0)

    @pl.when(g == 0)
    def _():
        react_scr[...] = h

    d = h - (g > 0).astype(f32) * react_scr[...]

    hp = _dot(d, pcm2W[:_H])
    msg = jnp.maximum(_dot(Gs, hp) + ep2, 0.0)
    agg = _dot(Gd, msg)
    d = jnp.maximum(_dot(d, upd2W[:_H]) + _dot(agg, upd2W[_H:])
                    + upd2b_ref[...], 0.0)

    gf = jnp.sum(d, axis=0, keepdims=True)
    s1 = jnp.maximum(_dot(gf, fc1W_ref[...]) + fc1b_ref[...], 0.0)
    score = _dot(s1, fc2W_ref[...]) + fc2b_ref[...]
    out_ref[pl.ds(g, 1), :] = jnp.broadcast_to(score, (1, 128))


def kernel(node_feats, edge_feats, candidate_scores, src, dst, graph_ids,
           pin_W, pin_b, pcm1_W, pcm1_b, upd1_W, upd1_b,
           pcm2_W, pcm2_b, upd2_W, upd2_b, fc1_W, fc1_b, fc2_W, fc2_b):
    f32 = jnp.float32
    offs = (jnp.arange(_NG, dtype=jnp.int32) * _NP).reshape(_NG, 1, 1)
    src_l = src.reshape(_NG, _EP, 1) - offs.reshape(_NG, 1, 1)
    dst_l = dst.reshape(_NG, 1, _EP) - offs
    nf = node_feats.reshape(_NG, _NP, _DN)
    ef = edge_feats.reshape(_NG, _EP, _DE)

    full = lambda shp: pl.BlockSpec(shp, lambda g: tuple(0 for _ in shp))
    in_specs = [
        pl.BlockSpec((1, _NP, _DN), lambda g: (g, 0, 0)),
        pl.BlockSpec((1, _EP, _DE), lambda g: (g, 0, 0)),
        pl.BlockSpec((1, _EP, 1), lambda g: (g, 0, 0)),
        pl.BlockSpec((1, 1, _EP), lambda g: (g, 0, 0)),
        full((_DN, _H)), full((1, _H)),
        full((_H + _DE, _H)), full((1, _H)),
        full((2 * _H, _H)), full((1, _H)),
        full((_H + _DE, _H)), full((1, _H)),
        full((2 * _H, _H)), full((1, _H)),
        full((_H, _H)), full((1, _H)),
        full((_H, 1)), full((1, 1)),
    ]

    out = pl.pallas_call(
        _body,
        grid=(_NG,),
        in_specs=in_specs,
        out_specs=pl.BlockSpec((_NG, 128), lambda g: (0, 0)),
        out_shape=jax.ShapeDtypeStruct((_NG, 128), f32),
        scratch_shapes=[pltpu.VMEM((_NP, _H), f32)],
        compiler_params=pltpu.CompilerParams(
            dimension_semantics=("arbitrary",)),
    )(nf, ef, src_l, dst_l,
      pin_W, pin_b.reshape(1, _H),
      pcm1_W, pcm1_b.reshape(1, _H), upd1_W, upd1_b.reshape(1, _H),
      pcm2_W, pcm2_b.reshape(1, _H), upd2_W, upd2_b.reshape(1, _H),
      fc1_W, fc1_b.reshape(1, _H), fc2_W, fc2_b.reshape(1, 1))

    return out[1:, :1] + candidate_scores
